# Initial kernel scaffold; baseline (speedup 1.0000x reference)
import jax
import jax.numpy as jnp
from jax.experimental import pallas as pl


def kernel(tokens, values, token_table, value_table):
    raise NotImplementedError("write your pallas kernel here")


# speedup vs baseline: 1.7663x; 1.7663x over previous
import functools

import jax
import jax.numpy as jnp
from jax import lax
from jax.experimental import pallas as pl
from jax.experimental.pallas import tpu as pltpu
from jax.experimental.pallas import tpu_sc as plsc

VOCAB = 1000000
EMBED = 32
B, L = 4096, 200
N = B * L

NC, NS, LANES = 2, 16, 16
NW = NC * NS
RPB = 128
NCH = 4
C = NCH * RPB
ROWS_TOTAL = N // RPB
ROWS_PER_W = ROWS_TOTAL // NW
NCHUNKS = ROWS_PER_W // NCH
GPB = RPB // LANES


def _sc_embed(tok2d, vals, tt, vt):
    mesh = plsc.VectorSubcoreMesh(core_axis_name="c", subcore_axis_name="s")

    @functools.partial(
        pl.kernel,
        mesh=mesh,
        compiler_params=pltpu.CompilerParams(use_tc_tiling_on_sc=False),
        out_type=jax.ShapeDtypeStruct((ROWS_TOTAL, RPB, EMBED), jnp.float32),
        scratch_types=[
            pltpu.VMEM((NCH, RPB), jnp.int32),
            pltpu.VMEM((C,), jnp.float32),
            pltpu.VMEM((NCH, RPB, EMBED), jnp.float32),
            pltpu.VMEM((NCH, RPB, EMBED), jnp.float32),
            pltpu.VMEM((NCH, RPB, EMBED), jnp.float32),
            pltpu.SemaphoreType.DMA,
        ],
    )
    def k(tok_hbm, val_hbm, tt_hbm, vt_hbm, out_hbm, idx_v, vals_v, trows, vrows, outv, sem):
        cid = lax.axis_index("c")
        sid = lax.axis_index("s")
        wid = sid * NC + cid
        row0 = wid * ROWS_PER_W

        def chunk_body(i, carry):
            rbase = row0 + i * NCH
            pltpu.sync_copy(tok_hbm.at[pl.ds(rbase, NCH)], idx_v)
            pltpu.sync_copy(val_hbm.at[pl.ds(rbase * RPB, C)], vals_v)
            copies = []
            for j in range(NCH):
                copies.append(pltpu.async_copy(tt_hbm.at[idx_v.at[j]], trows.at[j], sem))
                copies.append(pltpu.async_copy(vt_hbm.at[idx_v.at[j]], vrows.at[j], sem))
            for cp in copies:
                cp.wait()

            def group_body(g, gc):
                j = g // GPB
                r0 = (g % GPB) * LANES
                tokv = idx_v[j, pl.ds(r0, LANES)]
                valv = vals_v[pl.ds(g * LANES, LANES)]
                nz = tokv != 0
                m = jnp.where(nz, 1.0, 0.0)
                w = jnp.where(nz & (valv == valv), valv, 0.0)
                for lane in range(LANES):
                    mb = jnp.broadcast_to(m[lane], (LANES,))
                    wb = jnp.broadcast_to(w[lane], (LANES,))
                    rr = r0 + lane
                    t0 = trows[j, rr, pl.ds(0, LANES)]
                    t1 = trows[j, rr, pl.ds(LANES, LANES)]
                    v0 = vrows[j, rr, pl.ds(0, LANES)]
                    v1 = vrows[j, rr, pl.ds(LANES, LANES)]
                    outv[j, rr, pl.ds(0, LANES)] = t0 * mb + v0 * wb
                    outv[j, rr, pl.ds(LANES, LANES)] = t1 * mb + v1 * wb
                return gc

            lax.fori_loop(0, C // LANES, group_body, 0)
            pltpu.sync_copy(outv, out_hbm.at[pl.ds(rbase, NCH)])
            return carry

        lax.fori_loop(0, NCHUNKS, chunk_body, 0)

    return k(tok2d, vals, tt, vt)


def kernel(tokens, values, token_table, value_table):
    tok2d = tokens.reshape(ROWS_TOTAL, RPB).astype(jnp.int32)
    vals = values.reshape(N)
    out = _sc_embed(tok2d, vals, token_table, value_table)
    return out.reshape(B, L, EMBED)
